# baseline (speedup 1.0000x reference)
import functools

import jax
import jax.numpy as jnp
from jax import lax
from jax.experimental import pallas as pl
from jax.experimental.pallas import tpu as pltpu
from jax.experimental.pallas import tpu_sc as plsc

B = 4096
C = 128
P = 81
NC = 2
NS = 16
NW = NC * NS
RW = B // NW
CB = C // 16


def _h_body(hand_ref, w_ref, bt_ref, bo_ref, o_ref):
    o_ref[...] = (
        jnp.dot(hand_ref[...], w_ref[...], preferred_element_type=jnp.float32)
        + bt_ref[...] + bo_ref[...]
    )


def _hands_encode(hand, w, bt, bo):
    return pl.pallas_call(
        _h_body,
        out_shape=jax.ShapeDtypeStruct((B, C), jnp.float32),
    )(hand, w, bt, bo)


NB = 4
SLOT = P * NB * C
NG = RW // NB

_GATHER_DN = lax.GatherDimensionNumbers(
    offset_dims=(), collapsed_slice_dims=(0,), start_index_map=(0,)
)
XS = 128


def _sc_body(x_hbm, emb_hbm, h_hbm, out_hbm, emb_v, x_v, h_v, obuf, sem0, sem1):
    cid = lax.axis_index("c")
    sid = lax.axis_index("s")
    wid = sid * NC + cid
    base = wid * RW

    pltpu.sync_copy(emb_hbm, emb_v)
    pltpu.sync_copy(x_hbm.at[pl.ds(base * XS, RW * XS)], x_v)
    pltpu.sync_copy(h_hbm.at[pl.ds(base * C, RW * C)], h_v)

    lanes = lax.iota(jnp.int32, 16)
    sems = (sem0, sem1)

    def compute_row(i, slot, ii):
        hvecs = [
            plsc.load_gather(h_v, [i * C + cb * 16 + lanes]) for cb in range(CB)
        ]
        row_off = i * XS
        obuf_s = obuf.at[slot]
        iivec = jnp.full((16,), ii, jnp.int32)

        def do_pos(p, idx_splat):
            src = idx_splat * C + lanes
            pvec = jnp.full((16,), p, jnp.int32)
            for cb in range(CB):
                vals = plsc.load_gather(emb_v, [src + cb * 16])
                plsc.store_scatter(
                    obuf_s, [pvec, iivec, cb * 16 + lanes], vals + hvecs[cb]
                )

        @pl.loop(0, 5)
        def _grp(g):
            idx16 = x_v[pl.ds(row_off + g * 16, 16)]

            @plsc.parallel_loop(0, 16, unroll=3)
            def _pos(j):
                splat = lax.gather(
                    idx16,
                    jnp.full((16, 1), j, jnp.int32),
                    _GATHER_DN,
                    slice_sizes=(1,),
                    mode=lax.GatherScatterMode.PROMISE_IN_BOUNDS,
                )
                do_pos(g * 16 + j, splat)

        do_pos(
            P - 1,
            plsc.load_gather(x_v, [jnp.full((16,), row_off + P - 1, jnp.int32)]),
        )

    def group_copy(g, slot):
        return pltpu.make_async_copy(
            obuf.at[slot],
            out_hbm.at[:, pl.ds(base + g * NB, NB), :],
            sems[slot],
        )

    @pl.loop(0, NG // 2)
    def _groups(g2):
        for s in range(2):
            g = g2 * 2 + s

            @pl.when(g2 > 0)
            def _drain():
                group_copy(g - 2, s).wait()

            for ii in range(NB):
                compute_row(g * NB + ii, s, ii)
            group_copy(g, s).start()

    for s in range(2):
        group_copy(NG - 2 + s, s).wait()


@functools.partial(
    pl.kernel,
    out_type=jax.ShapeDtypeStruct((P, B, C), jnp.float32),
    mesh=plsc.VectorSubcoreMesh(
        core_axis_name="c", subcore_axis_name="s", num_cores=NC, num_subcores=NS
    ),
    compiler_params=pltpu.CompilerParams(needs_layout_passes=False),
    scratch_types=[
        pltpu.VMEM((88 * C,), jnp.float32),
        pltpu.VMEM((RW * XS,), jnp.int32),
        pltpu.VMEM((RW * C,), jnp.float32),
        pltpu.VMEM((2, P, NB, C), jnp.float32),
        pltpu.SemaphoreType.DMA,
        pltpu.SemaphoreType.DMA,
    ],
)
def _sc_gather_add(x_hbm, emb_hbm, h_hbm, out_hbm, emb_v, x_v, h_v, obuf, s0, s1):
    _sc_body(x_hbm, emb_hbm, h_hbm, out_hbm, emb_v, x_v, h_v, obuf, s0, s1)


def kernel(x, emb, Wt, bt, Wo, bo):
    x32 = x.astype(jnp.int32)
    hand = x32[:, 81:95].astype(jnp.float32)
    w = jnp.concatenate([Wt, Wo], axis=1).T
    h = _hands_encode(hand, w, bt, bo)
    xb = jnp.pad(x32[:, :P], ((0, 0), (0, XS - P)))
    out = _sc_gather_add(xb.reshape(-1), emb.reshape(-1), h.reshape(-1))
    return out.transpose(1, 0, 2)


# speedup vs baseline: 1.7517x; 1.7517x over previous
import functools

import jax
import jax.numpy as jnp
from jax import lax
from jax.experimental import pallas as pl
from jax.experimental.pallas import tpu as pltpu
from jax.experimental.pallas import tpu_sc as plsc

B = 4096
C = 128
P = 81
NC = 2
NS = 16
NW = NC * NS
RW = B // NW
CB = C // 16


def _prep_body(x_ref, w_ref, bt_ref, bo_ref, h_ref, xb_ref):
    xi = x_ref[...]
    hand = xi[:, 81:95].astype(jnp.float32)
    h_ref[...] = (
        jnp.dot(hand, w_ref[...], preferred_element_type=jnp.float32)
        + bt_ref[...] + bo_ref[...]
    )
    xb_ref[...] = jnp.pad(xi[:, :81], ((0, 0), (0, 128 - 81)))


def _tc_prep(x32, w, bt, bo):
    return pl.pallas_call(
        _prep_body,
        out_shape=(
            jax.ShapeDtypeStruct((B, C), jnp.float32),
            jax.ShapeDtypeStruct((B, 128), jnp.int32),
        ),
    )(x32, w, bt, bo)


NB = 4
SLOT = P * NB * C
NG = RW // NB
XS = 128


def _sc_body(x_hbm, emb_hbm, h_hbm, out_hbm, emb_v, x_v, h_v, obuf, sem0, sem1):
    cid = lax.axis_index("c")
    sid = lax.axis_index("s")
    wid = sid * NC + cid
    base = wid * RW

    cp0 = pltpu.async_copy(emb_hbm, emb_v, sem0)
    cp1 = pltpu.async_copy(x_hbm.at[pl.ds(base * XS, RW * XS)], x_v, sem1)
    cp2 = pltpu.async_copy(h_hbm.at[pl.ds(base * C, RW * C)], h_v, sem0)
    cp1.wait()
    cp0.wait()
    cp2.wait()

    lanes = lax.iota(jnp.int32, 16)
    sems = (sem0, sem1)

    def compute_row(i, slot, ii):
        hvecs = [
            plsc.load_gather(h_v, [i * C + cb * 16 + lanes]) for cb in range(CB)
        ]
        row_off = i * XS
        obuf_s = obuf.at[slot]
        iivec = jnp.full((16,), ii, jnp.int32)

        @plsc.parallel_loop(0, P, unroll=3)
        def _pos(p):
            idx = plsc.load_gather(x_v, [jnp.full((16,), row_off + p, jnp.int32)])
            src = idx * C + lanes
            pvec = jnp.full((16,), p, jnp.int32)
            for cb in range(CB):
                vals = plsc.load_gather(emb_v, [src + cb * 16])
                plsc.store_scatter(
                    obuf_s, [pvec, iivec, cb * 16 + lanes], vals + hvecs[cb]
                )

    def group_copy(g, slot):
        return pltpu.make_async_copy(
            obuf.at[slot],
            out_hbm.at[:, pl.ds(base + g * NB, NB), :],
            sems[slot],
        )

    @pl.loop(0, NG // 2)
    def _groups(g2):
        for s in range(2):
            g = g2 * 2 + s

            @pl.when(g2 > 0)
            def _drain():
                group_copy(g - 2, s).wait()

            for ii in range(NB):
                compute_row(g * NB + ii, s, ii)
            group_copy(g, s).start()

    for s in range(2):
        group_copy(NG - 2 + s, s).wait()


@functools.partial(
    pl.kernel,
    out_type=jax.ShapeDtypeStruct((P, B, C), jnp.float32),
    mesh=plsc.VectorSubcoreMesh(
        core_axis_name="c", subcore_axis_name="s", num_cores=NC, num_subcores=NS
    ),
    compiler_params=pltpu.CompilerParams(needs_layout_passes=False),
    scratch_types=[
        pltpu.VMEM((88 * C,), jnp.float32),
        pltpu.VMEM((RW * XS,), jnp.int32),
        pltpu.VMEM((RW * C,), jnp.float32),
        pltpu.VMEM((2, P, NB, C), jnp.float32),
        pltpu.SemaphoreType.DMA,
        pltpu.SemaphoreType.DMA,
    ],
)
def _sc_gather_add(x_hbm, emb_hbm, h_hbm, out_hbm, emb_v, x_v, h_v, obuf, s0, s1):
    _sc_body(x_hbm, emb_hbm, h_hbm, out_hbm, emb_v, x_v, h_v, obuf, s0, s1)


def kernel(x, emb, Wt, bt, Wo, bo):
    x32 = x.astype(jnp.int32)
    w = jnp.concatenate([Wt, Wo], axis=1).T
    h, xb = _tc_prep(x32, w, bt, bo)
    out = _sc_gather_add(xb.reshape(-1), emb.reshape(-1), h.reshape(-1))
    return out.transpose(1, 0, 2)
